# Initial kernel scaffold; baseline (speedup 1.0000x reference)
import jax
import jax.numpy as jnp
from jax.experimental import pallas as pl


def kernel(rna_data, tissue_id, tissue_table, seq_table, sec_table, loop_table, W1, b1, W2, b2, W3, b3):
    raise NotImplementedError("write your pallas kernel here")


# speedup vs baseline: 191.1315x; 191.1315x over previous
import jax
import jax.numpy as jnp
from jax.experimental import pallas as pl

B = 512
L = 2048
POOL = 128
WIN = 16
H = 128
VOCABS = (5, 4, 8)
MAX_NORM = 2.0


def _renorm(table):
    n = jnp.sqrt(jnp.sum(table * table, axis=1, keepdims=True))
    scale = jnp.minimum(1.0, MAX_NORM / jnp.maximum(n, 1e-7))
    return table * scale


def _make_body():
    def body(rna_ref, tis_ref, tistab_ref, seq_ref, sec_ref, loop_ref,
             w1h_ref, w1t_ref, w2_ref, w3t_ref,
             b1_ref, b2_ref, b3_ref, out_ref):
        bB = rna_ref.shape[0]
        acc = jnp.broadcast_to(b1_ref[:], (bB, H)).astype(jnp.float32)

        tid = tis_ref[:]
        oh = (tid == jax.lax.broadcasted_iota(jnp.int32, (bB, 29), 1)
              ).astype(jnp.float32)
        acc = acc + (oh @ _renorm(tistab_ref[:])) @ w1h_ref[:]

        tabs = (seq_ref, sec_ref, loop_ref)
        for k, V in enumerate(VOCABS):
            tab = _renorm(tabs[k][:]) * (1.0 / WIN)
            xk = rna_ref[:, k, :, :]
            for v in range(V):
                counts = jnp.sum((xk == v).astype(jnp.float32), axis=1)
                g = tab[v:v + 1, 0:1] * w1t_ref[k, 0]
                for d in range(1, 16):
                    g = g + tab[v:v + 1, d:d + 1] * w1t_ref[k, d]
                acc = acc + jax.lax.dot(counts, g,
                                        preferred_element_type=jnp.float32)

        h1 = jnp.maximum(acc, 0.0)
        h2 = jnp.maximum(h1 @ w2_ref[:] + b2_ref[:], 0.0)
        out_ref[:] = jnp.sum(h2 * w3t_ref[:], axis=1, keepdims=True) + b3_ref[:]
    return body


def kernel(rna_data, tissue_id, tissue_table, seq_table, sec_table, loop_table,
           W1, b1, W2, b2, W3, b3):
    rna_t = jnp.transpose(rna_data.reshape(B, POOL, WIN, 3), (0, 3, 2, 1))
    tis2 = tissue_id.reshape(B, 1)
    w1_head = W1[:16, :]
    w1t = jnp.transpose(W1[16:, :].reshape(POOL, 3, 16, H), (1, 2, 0, 3))

    bB = 128
    return pl.pallas_call(
        _make_body(),
        grid=(B // bB,),
        in_specs=[
            pl.BlockSpec((bB, 3, WIN, POOL), lambda i: (i, 0, 0, 0)),
            pl.BlockSpec((bB, 1), lambda i: (i, 0)),
            pl.BlockSpec((29, 16), lambda i: (0, 0)),
            pl.BlockSpec((5, 16), lambda i: (0, 0)),
            pl.BlockSpec((4, 16), lambda i: (0, 0)),
            pl.BlockSpec((8, 16), lambda i: (0, 0)),
            pl.BlockSpec((16, H), lambda i: (0, 0)),
            pl.BlockSpec((3, 16, POOL, H), lambda i: (0, 0, 0, 0)),
            pl.BlockSpec((H, 64), lambda i: (0, 0)),
            pl.BlockSpec((1, 64), lambda i: (0, 0)),
            pl.BlockSpec((1, H), lambda i: (0, 0)),
            pl.BlockSpec((1, 64), lambda i: (0, 0)),
            pl.BlockSpec((1, 1), lambda i: (0, 0)),
        ],
        out_specs=pl.BlockSpec((bB, 1), lambda i: (i, 0)),
        out_shape=jax.ShapeDtypeStruct((B, 1), jnp.float32),
    )(rna_t, tis2, tissue_table, seq_table, sec_table, loop_table,
      w1_head, w1t, W2, W3.reshape(1, 64),
      b1.reshape(1, H), b2.reshape(1, 64), b3.reshape(1, 1))
